# baseline (speedup 1.0000x reference)
import jax
import jax.numpy as jnp
from jax.experimental import pallas as pl
from jax.experimental.pallas import tpu as pltpu

SIZE = 262144
EMBED_DIM = 128
BATCH = 4096
ROWS = 4096


def _body(ptr_ref, q_ref, emb_ref, out_ref, optr_ref, emb_pad):
    pid = pl.program_id(0)
    s = pid * ROWS
    raw_ptr = ptr_ref[0]
    ptr = jnp.clip(raw_ptr, 0, SIZE - BATCH)

    optr_ref[0] = (raw_ptr + BATCH) % SIZE

    @pl.when(pid == 0)
    def _():
        emb_pad[pl.ds(ROWS, BATCH), :] = emb_ref[...]

    rows = s + jax.lax.broadcasted_iota(jnp.int32, (ROWS, EMBED_DIM), 0)
    in_upd = (rows >= ptr) & (rows < ptr + BATCH)
    start = jnp.clip(s - ptr + ROWS, 0, BATCH + ROWS)
    gathered = emb_pad[pl.ds(start, ROWS), :]
    out_ref[...] = jnp.where(in_upd, gathered, q_ref[...])


def kernel(embed_queue, queue_ptr, embeddings):
    grid = (SIZE // ROWS,)
    new_queue, new_ptr = pl.pallas_call(
        _body,
        grid=grid,
        in_specs=[
            pl.BlockSpec(memory_space=pltpu.SMEM),
            pl.BlockSpec((ROWS, EMBED_DIM), lambda i: (i, 0)),
            pl.BlockSpec((BATCH, EMBED_DIM), lambda i: (0, 0)),
        ],
        out_specs=[
            pl.BlockSpec((ROWS, EMBED_DIM), lambda i: (i, 0)),
            pl.BlockSpec(memory_space=pltpu.SMEM),
        ],
        out_shape=[
            jax.ShapeDtypeStruct((SIZE, EMBED_DIM), jnp.float32),
            jax.ShapeDtypeStruct((1,), jnp.int32),
        ],
        scratch_shapes=[pltpu.VMEM((BATCH + 2 * ROWS, EMBED_DIM), jnp.float32)],
        compiler_params=pltpu.CompilerParams(
            dimension_semantics=("arbitrary",),
        ),
    )(queue_ptr, embed_queue, embeddings)
    return new_queue, new_ptr
